# baseline (speedup 1.0000x reference)
import functools
import math
import jax
import jax.numpy as jnp
from jax import lax
from jax.experimental import pallas as pl
from jax.experimental.pallas import tpu as pltpu
from jax.experimental.pallas import tpu_sc as plsc

B, L, E, H = 1, 2048, 1024, 16
DH = E // H
LORA = DH // 2
RT = 256
NEG = -1e30

_DN_T = (((1,), (1,)), ((), ()))
_DN_N = (((1,), (0,)), ((), ()))


def _dot_t(a, b):
    return jax.lax.dot_general(a, b, _DN_T, preferred_element_type=jnp.float32)


def _dot_n(a, b):
    return jax.lax.dot_general(a, b, _DN_N, preferred_element_type=jnp.float32)


def _layernorm(x, g, b):
    mu = jnp.mean(x, axis=-1, keepdims=True)
    var = jnp.mean((x - mu) ** 2, axis=-1, keepdims=True)
    return (x - mu) / jnp.sqrt(var + 1e-5) * g + b


def _softmax(logits):
    m = jnp.max(logits, axis=-1, keepdims=True)
    p = jnp.exp(logits - m)
    return p / jnp.sum(p, axis=-1, keepdims=True)


def _pre_kernel(inp_ref, g_ref, b_ref, wqkv_ref, wg_ref, bg_ref,
                qkv_ref, gate_ref):
    x = _layernorm(inp_ref[...], g_ref[...], b_ref[...])
    qkv_ref[...] = _dot_t(x, wqkv_ref[...])
    gate_ref[...] = jax.nn.sigmoid(_dot_t(x, wg_ref[...]) + bg_ref[...])


def _pre(inp, ln_g, ln_b, W_qkv, W_gate, b_gate):
    return pl.pallas_call(
        _pre_kernel,
        grid=(L // RT,),
        in_specs=[
            pl.BlockSpec((RT, E), lambda i: (i, 0)),
            pl.BlockSpec((1, E), lambda i: (0, 0)),
            pl.BlockSpec((1, E), lambda i: (0, 0)),
            pl.BlockSpec((3 * E, E), lambda i: (0, 0)),
            pl.BlockSpec((3 * H, E), lambda i: (0, 0)),
            pl.BlockSpec((1, 3 * H), lambda i: (0, 0)),
        ],
        out_specs=[
            pl.BlockSpec((RT, 3 * E), lambda i: (i, 0)),
            pl.BlockSpec((RT, 3 * H), lambda i: (i, 0)),
        ],
        out_shape=[
            jax.ShapeDtypeStruct((L, 3 * E), jnp.float32),
            jax.ShapeDtypeStruct((L, 3 * H), jnp.float32),
        ],
    )(inp, ln_g.reshape(1, E), ln_b.reshape(1, E), W_qkv, W_gate,
      b_gate.reshape(1, 3 * H))


def _comp_scale(q1, kb, vb, wk1, bk1, wk2, bk2, wv1, bv1, wv2, bv2, blk):
    nblk = L // blk
    hk = _dot_t(kb, wk1) + bk1
    kc = _dot_t(hk * jax.nn.sigmoid(1.702 * hk), wk2) + bk2
    hv = _dot_t(vb, wv1) + bv1
    vc = _dot_t(hv * jax.nn.sigmoid(1.702 * hv), wv2) + bv2
    grp = jax.lax.broadcasted_iota(jnp.int32, (L, nblk), 0) // blk
    col = jax.lax.broadcasted_iota(jnp.int32, (L, nblk), 1)
    mask = jnp.where(col <= grp, 0.0, NEG)
    logits = _dot_t(q1, kc) / math.sqrt(LORA) + mask
    w = _softmax(logits)
    comp_out = _dot_n(w, vc)
    scores = jnp.sum(w, axis=0, keepdims=True)
    return comp_out, scores


def _nsa_kernel(q_ref, kba_ref, vba_ref, kbb_ref, vbb_ref,
                akw1, akb1, akw2, akb2, avw1, avb1, avw2, avb2,
                bkw1, bkb1, bkw2, bkb2, bvw1, bvb1, bvw2, bvb2,
                ca_ref, cb_ref, sca_ref, scb_ref):
    q1 = q_ref[0][:, :LORA]
    ca, sca = _comp_scale(q1, kba_ref[0], vba_ref[0],
                          akw1[...], akb1[...], akw2[...], akb2[...],
                          avw1[...], avb1[...], avw2[...], avb2[...], 32)
    cb, scb = _comp_scale(q1, kbb_ref[0], vbb_ref[0],
                          bkw1[...], bkb1[...], bkw2[...], bkb2[...],
                          bvw1[...], bvb1[...], bvw2[...], bvb2[...], 8)
    ca_ref[0] = ca
    cb_ref[0] = cb
    sca_ref[0] = sca
    scb_ref[0] = scb


def _nsa(q, kba, vba, kbb, vbb, wts):
    def head_spec(n, d):
        return pl.BlockSpec((1, n, d), lambda h: (h, 0, 0))

    def full_spec(shape):
        nd = len(shape)
        return pl.BlockSpec(shape, lambda h, _nd=nd: (0,) * _nd)

    in_specs = [
        head_spec(L, DH),
        head_spec(64, 32 * LORA), head_spec(64, 32 * LORA),
        head_spec(256, 8 * LORA), head_spec(256, 8 * LORA),
    ] + [full_spec(w.shape) for w in wts]
    return pl.pallas_call(
        _nsa_kernel,
        grid=(H,),
        in_specs=in_specs,
        out_specs=[head_spec(L, LORA), head_spec(L, LORA),
                   head_spec(1, 64), head_spec(1, 256)],
        out_shape=[jax.ShapeDtypeStruct((H, L, LORA), jnp.float32),
                   jax.ShapeDtypeStruct((H, L, LORA), jnp.float32),
                   jax.ShapeDtypeStruct((H, 1, 64), jnp.float32),
                   jax.ShapeDtypeStruct((H, 1, 256), jnp.float32)],
    )(q, kba, vba, kbb, vbb, *wts)


def _select(sca, scb, k2f, v2f):
    mesh = plsc.VectorSubcoreMesh(core_axis_name="c", subcore_axis_name="s")

    @functools.partial(
        pl.kernel, mesh=mesh,
        out_type=[
            jax.ShapeDtypeStruct((H, 16, 128), jnp.float32),
            jax.ShapeDtypeStruct((H, 16, 128), jnp.float32),
            jax.ShapeDtypeStruct((H, 16, 128), jnp.float32),
            jax.ShapeDtypeStruct((H, 16, 128), jnp.float32),
            jax.ShapeDtypeStruct((H, 16), jnp.int32),
            jax.ShapeDtypeStruct((H, 16), jnp.int32),
        ],
        scratch_types=[
            pltpu.VMEM((64,), jnp.float32),
            pltpu.VMEM((256,), jnp.float32),
            pltpu.VMEM((16,), jnp.int32),
            pltpu.VMEM((16, 128), jnp.float32),
            pltpu.VMEM((16, 128), jnp.float32),
            pltpu.VMEM((16,), jnp.int32),
            pltpu.SemaphoreType.DMA,
        ],
    )
    def sel_kernel(sa_hbm, sb_hbm, k2_hbm, v2_hbm,
                   ksa_hbm, vsa_hbm, ksb_hbm, vsb_hbm, ia_hbm, ib_hbm,
                   sa_v, sb_v, ridx_v, kr_v, vr_v, idx_v, sem):
        wid = lax.axis_index("s") * 2 + lax.axis_index("c")
        head = wid // 2
        scale = wid % 2
        iota = lax.iota(jnp.int32, 16)

        shuf_dn = lax.GatherDimensionNumbers(
            offset_dims=(), collapsed_slice_dims=(0,), start_index_map=(0,))

        def shuf(x, perm):
            return lax.gather(x, perm[:, None], shuf_dn, slice_sizes=(1,),
                              mode=lax.GatherScatterMode.PROMISE_IN_BOUNDS)

        def splat_max(x):
            for st in (1, 2, 4, 8):
                x = jnp.maximum(x, shuf(x, iota ^ st))
            return x

        def splat_min(x):
            for st in (1, 2, 4, 8):
                x = jnp.minimum(x, shuf(x, iota ^ st))
            return x

        def run(s_hbm, s_v, nblk, blk, topk, ks_hbm, vs_hbm, i_hbm):
            pltpu.sync_copy(s_hbm.at[head], s_v)
            nv = nblk // 16
            svs = [s_v[pl.ds(16 * j, 16)] for j in range(nv)]
            sel = jnp.zeros((16,), jnp.int32)
            inds = []
            for t in range(topk):
                vm = svs[0]
                for j in range(1, nv):
                    vm = jnp.maximum(vm, svs[j])
                mval = splat_max(vm)
                iv = jnp.full((16,), nblk, jnp.int32)
                for j in range(nv):
                    iv = jnp.minimum(
                        iv, jnp.where(svs[j] == mval, iota + 16 * j, nblk))
                ind = splat_min(iv)
                inds.append(ind)
                sel = jnp.where(iota == t, ind, sel)
                for j in range(nv):
                    svs[j] = jnp.where(iota + 16 * j == ind, NEG, svs[j])
            rpb = blk * LORA // 128
            slot = lax.shift_right_logical(iota, rpb.bit_length() - 1)
            rem = lax.bitwise_and(iota, rpb - 1)
            acc = jnp.zeros((16,), jnp.int32)
            for t in range(topk):
                acc = jnp.where(slot == t, inds[t] * rpb + rem, acc)
            ridx_v[...] = acc + head * (L * LORA // 128)
            idx_v[...] = sel
            pltpu.async_copy(k2_hbm.at[ridx_v], kr_v, sem).wait()
            pltpu.async_copy(v2_hbm.at[ridx_v], vr_v, sem).wait()
            pltpu.sync_copy(kr_v, ks_hbm.at[head])
            pltpu.sync_copy(vr_v, vs_hbm.at[head])
            pltpu.sync_copy(idx_v, i_hbm.at[head])

        @pl.when(scale == 0)
        def _():
            run(sa_hbm, sa_v, 64, 32, 2, ksa_hbm, vsa_hbm, ia_hbm)

        @pl.when(scale == 1)
        def _():
            run(sb_hbm, sb_v, 256, 8, 8, ksb_hbm, vsb_hbm, ib_hbm)

    return sel_kernel(sca, scb, k2f, v2f)


def _sel_scale(q2, ksel, vsel, idx_f, blk, roff):
    topk = 64 // blk
    qt = q2.shape[0]
    onehot = (jax.lax.broadcasted_iota(jnp.int32, (64, 16), 0) // blk ==
              jax.lax.broadcasted_iota(jnp.int32, (64, 16), 1)
              ).astype(jnp.float32)
    bid = _dot_t(idx_f, onehot)
    qgrp = ((jax.lax.broadcasted_iota(jnp.int32, (qt, 1), 0) + roff) // blk
            ).astype(jnp.float32)
    selmask = jnp.where(bid <= qgrp, 0.0, NEG)
    logits = _dot_t(q2, ksel) / math.sqrt(LORA) + selmask
    w = _softmax(logits)
    return _dot_n(w, vsel)


QT = 256


def _glob_kernel(q_ref, k_ref, v_ref, o_ref):
    i = pl.program_id(1)
    q = q_ref[0]
    row = jax.lax.broadcasted_iota(jnp.int32, (QT, QT), 0)
    col = jax.lax.broadcasted_iota(jnp.int32, (QT, QT), 1)

    def body(j, carry):
        m, l, acc = carry
        kt = k_ref[0, pl.ds(j * QT, QT), :]
        vt = v_ref[0, pl.ds(j * QT, QT), :]
        s = _dot_t(q, kt) / math.sqrt(DH)
        s = jnp.where(col + (j - i) * QT <= row, s, NEG)
        m2 = jnp.maximum(m, jnp.max(s, axis=-1, keepdims=True))
        p = jnp.exp(s - m2)
        sc = jnp.exp(m - m2)
        l2 = l * sc + jnp.sum(p, axis=-1, keepdims=True)
        acc2 = acc * sc + _dot_n(p, vt)
        return m2, l2, acc2

    m0 = jnp.full((QT, 1), NEG, jnp.float32)
    l0 = jnp.zeros((QT, 1), jnp.float32)
    a0 = jnp.zeros((QT, DH), jnp.float32)
    m, l, acc = jax.lax.fori_loop(0, i + 1, body, (m0, l0, a0))
    o_ref[0] = acc / l


def _glob(qkvh):
    return pl.pallas_call(
        _glob_kernel,
        grid=(H, L // QT),
        in_specs=[
            pl.BlockSpec((1, QT, DH), lambda h, i: (h, i, 0)),
            pl.BlockSpec((1, L, DH), lambda h, i: (H + h, 0, 0)),
            pl.BlockSpec((1, L, DH), lambda h, i: (2 * H + h, 0, 0)),
        ],
        out_specs=pl.BlockSpec((1, QT, DH), lambda h, i: (h, i, 0)),
        out_shape=jax.ShapeDtypeStruct((H, L, DH), jnp.float32),
    )(qkvh, qkvh, qkvh)


def _sel_att_kernel(q_ref, ksa_ref, vsa_ref, ksb_ref, vsb_ref,
                    ia_ref, ib_ref, sa_ref, sb_ref):
    q2 = q_ref[0][:, LORA:]
    sa_ref[0] = _sel_scale(q2, ksa_ref[0], vsa_ref[0],
                           ia_ref[0].astype(jnp.float32), 32, 0)
    sb_ref[0] = _sel_scale(q2, ksb_ref[0], vsb_ref[0],
                           ib_ref[0].astype(jnp.float32), 8, 0)


def _sel_att(q, ksa, vsa, ksb, vsb, ia, ib):
    def head_spec(n, d):
        return pl.BlockSpec((1, n, d), lambda h: (h, 0, 0))

    return pl.pallas_call(
        _sel_att_kernel,
        grid=(H,),
        in_specs=[head_spec(L, DH),
                  head_spec(64, LORA), head_spec(64, LORA),
                  head_spec(64, LORA), head_spec(64, LORA),
                  head_spec(1, 16), head_spec(1, 16)],
        out_specs=[head_spec(L, LORA), head_spec(L, LORA)],
        out_shape=[jax.ShapeDtypeStruct((H, L, LORA), jnp.float32),
                   jax.ShapeDtypeStruct((H, L, LORA), jnp.float32)],
    )(q, ksa, vsa, ksb, vsb, ia, ib)


def _merge_kernel(ca_ref, sa_ref, cb_ref, sb_ref, glb_ref, gate_ref,
                  g_ref, b_ref, wout_ref, o_ref):
    gt = gate_ref[...]
    pieces = []
    for h in range(H):
        g0 = gt[:, 3 * h:3 * h + 1]
        g1 = gt[:, 3 * h + 1:3 * h + 2]
        g2 = gt[:, 3 * h + 2:3 * h + 3]
        o1 = jnp.concatenate([ca_ref[h], sa_ref[h]], axis=1)
        o2 = jnp.concatenate([cb_ref[h], sb_ref[h]], axis=1)
        pieces.append(g0 * o1 + g1 * o2 + g2 * glb_ref[h])
    y = jnp.concatenate(pieces, axis=1)
    y = _layernorm(y, g_ref[...], b_ref[...])
    o_ref[...] = _dot_t(y, wout_ref[...])


def _merge(ca, sa, cb, sb, glb, gate, ln_g, ln_b, W_out):
    return pl.pallas_call(
        _merge_kernel,
        grid=(L // RT,),
        in_specs=[
            pl.BlockSpec((H, RT, LORA), lambda i: (0, i, 0)),
            pl.BlockSpec((H, RT, LORA), lambda i: (0, i, 0)),
            pl.BlockSpec((H, RT, LORA), lambda i: (0, i, 0)),
            pl.BlockSpec((H, RT, LORA), lambda i: (0, i, 0)),
            pl.BlockSpec((H, RT, DH), lambda i: (0, i, 0)),
            pl.BlockSpec((RT, 3 * H), lambda i: (i, 0)),
            pl.BlockSpec((1, E), lambda i: (0, 0)),
            pl.BlockSpec((1, E), lambda i: (0, 0)),
            pl.BlockSpec((E, E), lambda i: (0, 0)),
        ],
        out_specs=pl.BlockSpec((RT, E), lambda i: (i, 0)),
        out_shape=jax.ShapeDtypeStruct((L, E), jnp.float32),
    )(ca, sa, cb, sb, glb, gate, ln_g.reshape(1, E), ln_b.reshape(1, E),
      W_out)


def kernel(inp, ln_g, ln_b, W_qkv,
           a_kW1, a_kb1, a_kW2, a_kb2, a_vW1, a_vb1, a_vW2, a_vb2,
           b_kW1, b_kb1, b_kW2, b_kb2, b_vW1, b_vb1, b_vW2, b_vb2,
           W_gate, b_gate, W_out):
    qkv, gate = _pre(inp[0], ln_g, ln_b, W_qkv, W_gate, b_gate)
    qkvh = qkv.reshape(L, 3 * H, DH).transpose(1, 0, 2)
    k1 = qkvh[H:2 * H, :, :LORA]
    v1 = qkvh[2 * H:, :, :LORA]
    kba = k1.reshape(H, 64, 32 * LORA)
    vba = v1.reshape(H, 64, 32 * LORA)
    kbb = k1.reshape(H, 256, 8 * LORA)
    vbb = v1.reshape(H, 256, 8 * LORA)
    wts = (a_kW1, a_kb1.reshape(1, LORA), a_kW2, a_kb2.reshape(1, LORA),
           a_vW1, a_vb1.reshape(1, LORA), a_vW2, a_vb2.reshape(1, LORA),
           b_kW1, b_kb1.reshape(1, LORA), b_kW2, b_kb2.reshape(1, LORA),
           b_vW1, b_vb1.reshape(1, LORA), b_vW2, b_vb2.reshape(1, LORA))
    ca, cb, sca, scb = _nsa(qkvh, kba, vba, kbb, vbb, wts)
    k2f = qkvh[H:2 * H, :, LORA:].reshape(H * L * LORA // 128, 128)
    v2f = qkvh[2 * H:, :, LORA:].reshape(H * L * LORA // 128, 128)
    ksa, vsa, ksb, vsb, ia, ib = _select(
        sca.reshape(H, 64), scb.reshape(H, 256), k2f, v2f)
    glb = _glob(qkvh)
    sa, sb = _sel_att(qkvh, ksa.reshape(H, 64, LORA), vsa.reshape(H, 64, LORA),
                      ksb.reshape(H, 64, LORA), vsb.reshape(H, 64, LORA),
                      ia.reshape(H, 1, 16), ib.reshape(H, 1, 16))
    out = _merge(ca, sa, cb, sb, glb, gate, ln_g, ln_b, W_out)
    return out.reshape(B, L, E)


# speedup vs baseline: 1.3721x; 1.3721x over previous
import functools
import math
import jax
import jax.numpy as jnp
from jax import lax
from jax.experimental import pallas as pl
from jax.experimental.pallas import tpu as pltpu
from jax.experimental.pallas import tpu_sc as plsc

B, L, E, H = 1, 2048, 1024, 16
DH = E // H
LORA = DH // 2
RT = 256
NEG = -1e30

_DN_T = (((1,), (1,)), ((), ()))
_DN_N = (((1,), (0,)), ((), ()))


def _dot_t(a, b):
    return jax.lax.dot_general(a, b, _DN_T, preferred_element_type=jnp.float32)


def _dot_n(a, b):
    return jax.lax.dot_general(a, b, _DN_N, preferred_element_type=jnp.float32)


def _layernorm(x, g, b):
    mu = jnp.mean(x, axis=-1, keepdims=True)
    var = jnp.mean((x - mu) ** 2, axis=-1, keepdims=True)
    return (x - mu) / jnp.sqrt(var + 1e-5) * g + b


def _softmax(logits):
    m = jnp.max(logits, axis=-1, keepdims=True)
    p = jnp.exp(logits - m)
    return p / jnp.sum(p, axis=-1, keepdims=True)


def _pre_kernel(inp_ref, g_ref, b_ref, wqkv_ref, wg_ref, bg_ref,
                qkv_ref, gate_ref):
    x = _layernorm(inp_ref[...], g_ref[...], b_ref[...])
    qkv_ref[...] = _dot_t(x, wqkv_ref[...])
    gate_ref[...] = jax.nn.sigmoid(_dot_t(x, wg_ref[...]) + bg_ref[...])


def _pre(inp, ln_g, ln_b, W_qkv, W_gate, b_gate):
    return pl.pallas_call(
        _pre_kernel,
        grid=(L // RT,),
        in_specs=[
            pl.BlockSpec((RT, E), lambda i: (i, 0)),
            pl.BlockSpec((1, E), lambda i: (0, 0)),
            pl.BlockSpec((1, E), lambda i: (0, 0)),
            pl.BlockSpec((3 * E, E), lambda i: (0, 0)),
            pl.BlockSpec((3 * H, E), lambda i: (0, 0)),
            pl.BlockSpec((1, 3 * H), lambda i: (0, 0)),
        ],
        out_specs=[
            pl.BlockSpec((RT, 3 * E), lambda i: (i, 0)),
            pl.BlockSpec((RT, 3 * H), lambda i: (i, 0)),
        ],
        out_shape=[
            jax.ShapeDtypeStruct((L, 3 * E), jnp.float32),
            jax.ShapeDtypeStruct((L, 3 * H), jnp.float32),
        ],
    )(inp, ln_g.reshape(1, E), ln_b.reshape(1, E), W_qkv, W_gate,
      b_gate.reshape(1, 3 * H))


def _comp_scale(q1, kb, vb, wk1, bk1, wk2, bk2, wv1, bv1, wv2, bv2, blk):
    nblk = L // blk
    hk = _dot_t(kb, wk1) + bk1
    kc = _dot_t(hk * jax.nn.sigmoid(1.702 * hk), wk2) + bk2
    hv = _dot_t(vb, wv1) + bv1
    vc = _dot_t(hv * jax.nn.sigmoid(1.702 * hv), wv2) + bv2
    grp = jax.lax.broadcasted_iota(jnp.int32, (L, nblk), 0) // blk
    col = jax.lax.broadcasted_iota(jnp.int32, (L, nblk), 1)
    mask = jnp.where(col <= grp, 0.0, NEG)
    logits = _dot_t(q1, kc) / math.sqrt(LORA) + mask
    w = _softmax(logits)
    comp_out = _dot_n(w, vc)
    scores = jnp.sum(w, axis=0, keepdims=True)
    return comp_out, scores


def _nsa_kernel(q_ref, kba_ref, vba_ref, kbb_ref, vbb_ref,
                akw1, akb1, akw2, akb2, avw1, avb1, avw2, avb2,
                bkw1, bkb1, bkw2, bkb2, bvw1, bvb1, bvw2, bvb2,
                ca_ref, cb_ref, sca_ref, scb_ref):
    q1 = q_ref[0][:, :LORA]
    ca, sca = _comp_scale(q1, kba_ref[0], vba_ref[0],
                          akw1[...], akb1[...], akw2[...], akb2[...],
                          avw1[...], avb1[...], avw2[...], avb2[...], 32)
    cb, scb = _comp_scale(q1, kbb_ref[0], vbb_ref[0],
                          bkw1[...], bkb1[...], bkw2[...], bkb2[...],
                          bvw1[...], bvb1[...], bvw2[...], bvb2[...], 8)
    ca_ref[0] = ca
    cb_ref[0] = cb
    sca_ref[0] = sca
    scb_ref[0] = scb


def _nsa(q, kba, vba, kbb, vbb, wts):
    def head_spec(n, d):
        return pl.BlockSpec((1, n, d), lambda h: (h, 0, 0))

    def full_spec(shape):
        nd = len(shape)
        return pl.BlockSpec(shape, lambda h, _nd=nd: (0,) * _nd)

    in_specs = [
        head_spec(L, DH),
        head_spec(64, 32 * LORA), head_spec(64, 32 * LORA),
        head_spec(256, 8 * LORA), head_spec(256, 8 * LORA),
    ] + [full_spec(w.shape) for w in wts]
    return pl.pallas_call(
        _nsa_kernel,
        grid=(H,),
        in_specs=in_specs,
        out_specs=[head_spec(L, LORA), head_spec(L, LORA),
                   head_spec(1, 64), head_spec(1, 256)],
        out_shape=[jax.ShapeDtypeStruct((H, L, LORA), jnp.float32),
                   jax.ShapeDtypeStruct((H, L, LORA), jnp.float32),
                   jax.ShapeDtypeStruct((H, 1, 64), jnp.float32),
                   jax.ShapeDtypeStruct((H, 1, 256), jnp.float32)],
    )(q, kba, vba, kbb, vbb, *wts)


def _select(sca, scb, k2f, v2f):
    mesh = plsc.VectorSubcoreMesh(core_axis_name="c", subcore_axis_name="s")

    @functools.partial(
        pl.kernel, mesh=mesh,
        out_type=[
            jax.ShapeDtypeStruct((H, 16, 128), jnp.float32),
            jax.ShapeDtypeStruct((H, 16, 128), jnp.float32),
            jax.ShapeDtypeStruct((H, 16, 128), jnp.float32),
            jax.ShapeDtypeStruct((H, 16, 128), jnp.float32),
            jax.ShapeDtypeStruct((H, 16), jnp.int32),
            jax.ShapeDtypeStruct((H, 16), jnp.int32),
        ],
        scratch_types=[
            pltpu.VMEM((64,), jnp.float32),
            pltpu.VMEM((256,), jnp.float32),
            pltpu.VMEM((16,), jnp.int32),
            pltpu.VMEM((16, 128), jnp.float32),
            pltpu.VMEM((16, 128), jnp.float32),
            pltpu.VMEM((16,), jnp.int32),
            pltpu.SemaphoreType.DMA,
        ],
    )
    def sel_kernel(sa_hbm, sb_hbm, k2_hbm, v2_hbm,
                   ksa_hbm, vsa_hbm, ksb_hbm, vsb_hbm, ia_hbm, ib_hbm,
                   sa_v, sb_v, ridx_v, kr_v, vr_v, idx_v, sem):
        wid = lax.axis_index("s") * 2 + lax.axis_index("c")
        head = wid // 2
        scale = wid % 2
        iota = lax.iota(jnp.int32, 16)

        shuf_dn = lax.GatherDimensionNumbers(
            offset_dims=(), collapsed_slice_dims=(0,), start_index_map=(0,))

        def shuf(x, perm):
            return lax.gather(x, perm[:, None], shuf_dn, slice_sizes=(1,),
                              mode=lax.GatherScatterMode.PROMISE_IN_BOUNDS)

        def splat_max(x):
            for st in (1, 2, 4, 8):
                x = jnp.maximum(x, shuf(x, iota ^ st))
            return x

        def splat_min(x):
            for st in (1, 2, 4, 8):
                x = jnp.minimum(x, shuf(x, iota ^ st))
            return x

        def run(s_hbm, s_v, nblk, blk, topk, ks_hbm, vs_hbm, i_hbm):
            pltpu.sync_copy(s_hbm.at[head], s_v)
            nv = nblk // 16
            svs = [s_v[pl.ds(16 * j, 16)] for j in range(nv)]
            sel = jnp.zeros((16,), jnp.int32)
            inds = []
            for t in range(topk):
                vm = svs[0]
                for j in range(1, nv):
                    vm = jnp.maximum(vm, svs[j])
                mval = splat_max(vm)
                iv = jnp.full((16,), nblk, jnp.int32)
                for j in range(nv):
                    iv = jnp.minimum(
                        iv, jnp.where(svs[j] == mval, iota + 16 * j, nblk))
                ind = splat_min(iv)
                inds.append(ind)
                sel = jnp.where(iota == t, ind, sel)
                for j in range(nv):
                    svs[j] = jnp.where(iota + 16 * j == ind, NEG, svs[j])
            rpb = blk * LORA // 128
            slot = lax.shift_right_logical(iota, rpb.bit_length() - 1)
            rem = lax.bitwise_and(iota, rpb - 1)
            acc = jnp.zeros((16,), jnp.int32)
            for t in range(topk):
                acc = jnp.where(slot == t, inds[t] * rpb + rem, acc)
            ridx_v[...] = acc + head * (L * LORA // 128)
            idx_v[...] = sel
            pltpu.async_copy(k2_hbm.at[ridx_v], kr_v, sem).wait()
            pltpu.async_copy(v2_hbm.at[ridx_v], vr_v, sem).wait()
            pltpu.sync_copy(kr_v, ks_hbm.at[head])
            pltpu.sync_copy(vr_v, vs_hbm.at[head])
            pltpu.sync_copy(idx_v, i_hbm.at[head])

        @pl.when(scale == 0)
        def _():
            run(sa_hbm, sa_v, 64, 32, 2, ksa_hbm, vsa_hbm, ia_hbm)

        @pl.when(scale == 1)
        def _():
            run(sb_hbm, sb_v, 256, 8, 8, ksb_hbm, vsb_hbm, ib_hbm)

    return sel_kernel(sca, scb, k2f, v2f)


def _sel_scale(q2, ksel, vsel, idx_f, blk, roff):
    topk = 64 // blk
    qt = q2.shape[0]
    onehot = (jax.lax.broadcasted_iota(jnp.int32, (64, 16), 0) // blk ==
              jax.lax.broadcasted_iota(jnp.int32, (64, 16), 1)
              ).astype(jnp.float32)
    bid = _dot_t(idx_f, onehot)
    qgrp = ((jax.lax.broadcasted_iota(jnp.int32, (qt, 1), 0) + roff) // blk
            ).astype(jnp.float32)
    selmask = jnp.where(bid <= qgrp, 0.0, NEG)
    logits = _dot_t(q2, ksel) / math.sqrt(LORA) + selmask
    w = _softmax(logits)
    return _dot_n(w, vsel)


QT = 1024


def _glob_kernel(q_ref, k_ref, v_ref, o_ref):
    i = pl.program_id(1)
    q = q_ref[0]
    row = jax.lax.broadcasted_iota(jnp.int32, (QT, QT), 0)
    col = jax.lax.broadcasted_iota(jnp.int32, (QT, QT), 1)

    def body(j, carry):
        m, l, acc = carry
        kt = k_ref[0, pl.ds(j * QT, QT), :]
        vt = v_ref[0, pl.ds(j * QT, QT), :]
        s = _dot_t(q, kt) / math.sqrt(DH)
        s = jnp.where(col + (j - i) * QT <= row, s, NEG)
        m2 = jnp.maximum(m, jnp.max(s, axis=-1, keepdims=True))
        p = jnp.exp(s - m2)
        sc = jnp.exp(m - m2)
        l2 = l * sc + jnp.sum(p, axis=-1, keepdims=True)
        acc2 = acc * sc + _dot_n(p, vt)
        return m2, l2, acc2

    m0 = jnp.full((QT, 1), NEG, jnp.float32)
    l0 = jnp.zeros((QT, 1), jnp.float32)
    a0 = jnp.zeros((QT, DH), jnp.float32)
    m, l, acc = jax.lax.fori_loop(0, i + 1, body, (m0, l0, a0))
    o_ref[0] = acc / l


def _glob(qkvh):
    return pl.pallas_call(
        _glob_kernel,
        grid=(H, L // QT),
        in_specs=[
            pl.BlockSpec((1, QT, DH), lambda h, i: (h, i, 0)),
            pl.BlockSpec((1, L, DH), lambda h, i: (H + h, 0, 0)),
            pl.BlockSpec((1, L, DH), lambda h, i: (2 * H + h, 0, 0)),
        ],
        out_specs=pl.BlockSpec((1, QT, DH), lambda h, i: (h, i, 0)),
        out_shape=jax.ShapeDtypeStruct((H, L, DH), jnp.float32),
    )(qkvh, qkvh, qkvh)


def _sel_att_kernel(q_ref, ksa_ref, vsa_ref, ksb_ref, vsb_ref,
                    ia_ref, ib_ref, sa_ref, sb_ref):
    q2 = q_ref[0][:, LORA:]
    sa_ref[0] = _sel_scale(q2, ksa_ref[0], vsa_ref[0],
                           ia_ref[0].astype(jnp.float32), 32, 0)
    sb_ref[0] = _sel_scale(q2, ksb_ref[0], vsb_ref[0],
                           ib_ref[0].astype(jnp.float32), 8, 0)


def _sel_att(q, ksa, vsa, ksb, vsb, ia, ib):
    def head_spec(n, d):
        return pl.BlockSpec((1, n, d), lambda h: (h, 0, 0))

    return pl.pallas_call(
        _sel_att_kernel,
        grid=(H,),
        in_specs=[head_spec(L, DH),
                  head_spec(64, LORA), head_spec(64, LORA),
                  head_spec(64, LORA), head_spec(64, LORA),
                  head_spec(1, 16), head_spec(1, 16)],
        out_specs=[head_spec(L, LORA), head_spec(L, LORA)],
        out_shape=[jax.ShapeDtypeStruct((H, L, LORA), jnp.float32),
                   jax.ShapeDtypeStruct((H, L, LORA), jnp.float32)],
    )(q, ksa, vsa, ksb, vsb, ia, ib)


def _merge_kernel(ca_ref, sa_ref, cb_ref, sb_ref, glb_ref, gate_ref,
                  g_ref, b_ref, wout_ref, o_ref):
    gt = gate_ref[...]
    pieces = []
    for h in range(H):
        g0 = gt[:, 3 * h:3 * h + 1]
        g1 = gt[:, 3 * h + 1:3 * h + 2]
        g2 = gt[:, 3 * h + 2:3 * h + 3]
        o1 = jnp.concatenate([ca_ref[h], sa_ref[h]], axis=1)
        o2 = jnp.concatenate([cb_ref[h], sb_ref[h]], axis=1)
        pieces.append(g0 * o1 + g1 * o2 + g2 * glb_ref[h])
    y = jnp.concatenate(pieces, axis=1)
    y = _layernorm(y, g_ref[...], b_ref[...])
    o_ref[...] = _dot_t(y, wout_ref[...])


def _merge(ca, sa, cb, sb, glb, gate, ln_g, ln_b, W_out):
    return pl.pallas_call(
        _merge_kernel,
        grid=(L // RT,),
        in_specs=[
            pl.BlockSpec((H, RT, LORA), lambda i: (0, i, 0)),
            pl.BlockSpec((H, RT, LORA), lambda i: (0, i, 0)),
            pl.BlockSpec((H, RT, LORA), lambda i: (0, i, 0)),
            pl.BlockSpec((H, RT, LORA), lambda i: (0, i, 0)),
            pl.BlockSpec((H, RT, DH), lambda i: (0, i, 0)),
            pl.BlockSpec((RT, 3 * H), lambda i: (i, 0)),
            pl.BlockSpec((1, E), lambda i: (0, 0)),
            pl.BlockSpec((1, E), lambda i: (0, 0)),
            pl.BlockSpec((E, E), lambda i: (0, 0)),
        ],
        out_specs=pl.BlockSpec((RT, E), lambda i: (i, 0)),
        out_shape=jax.ShapeDtypeStruct((L, E), jnp.float32),
    )(ca, sa, cb, sb, glb, gate, ln_g.reshape(1, E), ln_b.reshape(1, E),
      W_out)


def kernel(inp, ln_g, ln_b, W_qkv,
           a_kW1, a_kb1, a_kW2, a_kb2, a_vW1, a_vb1, a_vW2, a_vb2,
           b_kW1, b_kb1, b_kW2, b_kb2, b_vW1, b_vb1, b_vW2, b_vb2,
           W_gate, b_gate, W_out):
    qkv, gate = _pre(inp[0], ln_g, ln_b, W_qkv, W_gate, b_gate)
    qkvh = qkv.reshape(L, 3 * H, DH).transpose(1, 0, 2)
    k1 = qkvh[H:2 * H, :, :LORA]
    v1 = qkvh[2 * H:, :, :LORA]
    kba = k1.reshape(H, 64, 32 * LORA)
    vba = v1.reshape(H, 64, 32 * LORA)
    kbb = k1.reshape(H, 256, 8 * LORA)
    vbb = v1.reshape(H, 256, 8 * LORA)
    wts = (a_kW1, a_kb1.reshape(1, LORA), a_kW2, a_kb2.reshape(1, LORA),
           a_vW1, a_vb1.reshape(1, LORA), a_vW2, a_vb2.reshape(1, LORA),
           b_kW1, b_kb1.reshape(1, LORA), b_kW2, b_kb2.reshape(1, LORA),
           b_vW1, b_vb1.reshape(1, LORA), b_vW2, b_vb2.reshape(1, LORA))
    ca, cb, sca, scb = _nsa(qkvh, kba, vba, kbb, vbb, wts)
    k2f = qkvh[H:2 * H, :, LORA:].reshape(H * L * LORA // 128, 128)
    v2f = qkvh[2 * H:, :, LORA:].reshape(H * L * LORA // 128, 128)
    ksa, vsa, ksb, vsb, ia, ib = _select(
        sca.reshape(H, 64), scb.reshape(H, 256), k2f, v2f)
    glb = _glob(qkvh)
    sa, sb = _sel_att(qkvh, ksa.reshape(H, 64, LORA), vsa.reshape(H, 64, LORA),
                      ksb.reshape(H, 64, LORA), vsb.reshape(H, 64, LORA),
                      ia.reshape(H, 1, 16), ib.reshape(H, 1, 16))
    out = _merge(ca, sa, cb, sb, glb, gate, ln_g, ln_b, W_out)
    return out.reshape(B, L, E)


# speedup vs baseline: 1.3948x; 1.0165x over previous
import functools
import math
import jax
import jax.numpy as jnp
from jax import lax
from jax.experimental import pallas as pl
from jax.experimental.pallas import tpu as pltpu
from jax.experimental.pallas import tpu_sc as plsc

B, L, E, H = 1, 2048, 1024, 16
DH = E // H
LORA = DH // 2
RT = 256
NEG = -1e30

_DN_T = (((1,), (1,)), ((), ()))
_DN_N = (((1,), (0,)), ((), ()))


def _dot_t(a, b):
    return jax.lax.dot_general(a, b, _DN_T, preferred_element_type=jnp.float32)


def _dot_n(a, b):
    return jax.lax.dot_general(a, b, _DN_N, preferred_element_type=jnp.float32)


def _layernorm(x, g, b):
    mu = jnp.mean(x, axis=-1, keepdims=True)
    var = jnp.mean((x - mu) ** 2, axis=-1, keepdims=True)
    return (x - mu) / jnp.sqrt(var + 1e-5) * g + b


def _softmax(logits):
    m = jnp.max(logits, axis=-1, keepdims=True)
    p = jnp.exp(logits - m)
    return p / jnp.sum(p, axis=-1, keepdims=True)


def _pre_kernel(inp_ref, g_ref, b_ref, wqkv_ref, wg_ref, bg_ref,
                qkv_ref, gate_ref):
    x = _layernorm(inp_ref[...], g_ref[...], b_ref[...])
    qkv_ref[...] = _dot_t(x, wqkv_ref[...])
    gate_ref[...] = jax.nn.sigmoid(_dot_t(x, wg_ref[...]) + bg_ref[...])


def _pre(inp, ln_g, ln_b, W_qkv, W_gate, b_gate):
    return pl.pallas_call(
        _pre_kernel,
        grid=(L // RT,),
        in_specs=[
            pl.BlockSpec((RT, E), lambda i: (i, 0)),
            pl.BlockSpec((1, E), lambda i: (0, 0)),
            pl.BlockSpec((1, E), lambda i: (0, 0)),
            pl.BlockSpec((3 * E, E), lambda i: (0, 0)),
            pl.BlockSpec((3 * H, E), lambda i: (0, 0)),
            pl.BlockSpec((1, 3 * H), lambda i: (0, 0)),
        ],
        out_specs=[
            pl.BlockSpec((RT, 3 * E), lambda i: (i, 0)),
            pl.BlockSpec((RT, 3 * H), lambda i: (i, 0)),
        ],
        out_shape=[
            jax.ShapeDtypeStruct((L, 3 * E), jnp.float32),
            jax.ShapeDtypeStruct((L, 3 * H), jnp.float32),
        ],
    )(inp, ln_g.reshape(1, E), ln_b.reshape(1, E), W_qkv, W_gate,
      b_gate.reshape(1, 3 * H))


def _comp_scale(q1, kb, vb, wk1, bk1, wk2, bk2, wv1, bv1, wv2, bv2, blk):
    nblk = L // blk
    hk = _dot_t(kb, wk1) + bk1
    kc = _dot_t(hk * jax.nn.sigmoid(1.702 * hk), wk2) + bk2
    hv = _dot_t(vb, wv1) + bv1
    vc = _dot_t(hv * jax.nn.sigmoid(1.702 * hv), wv2) + bv2
    grp = jax.lax.broadcasted_iota(jnp.int32, (L, nblk), 0) // blk
    col = jax.lax.broadcasted_iota(jnp.int32, (L, nblk), 1)
    mask = jnp.where(col <= grp, 0.0, NEG)
    logits = _dot_t(q1, kc) / math.sqrt(LORA) + mask
    w = _softmax(logits)
    comp_out = _dot_n(w, vc)
    scores = jnp.sum(w, axis=0, keepdims=True)
    return comp_out, scores


def _nsa_kernel(q_ref, kba_ref, vba_ref, kbb_ref, vbb_ref,
                akw1, akb1, akw2, akb2, avw1, avb1, avw2, avb2,
                bkw1, bkb1, bkw2, bkb2, bvw1, bvb1, bvw2, bvb2,
                ca_ref, cb_ref, sca_ref, scb_ref):
    q1 = q_ref[0][:, :LORA]
    ca, sca = _comp_scale(q1, kba_ref[0], vba_ref[0],
                          akw1[...], akb1[...], akw2[...], akb2[...],
                          avw1[...], avb1[...], avw2[...], avb2[...], 32)
    cb, scb = _comp_scale(q1, kbb_ref[0], vbb_ref[0],
                          bkw1[...], bkb1[...], bkw2[...], bkb2[...],
                          bvw1[...], bvb1[...], bvw2[...], bvb2[...], 8)
    ca_ref[0] = ca
    cb_ref[0] = cb
    sca_ref[0] = sca
    scb_ref[0] = scb


def _nsa(q, kba, vba, kbb, vbb, wts):
    def head_spec(n, d):
        return pl.BlockSpec((1, n, d), lambda h: (h, 0, 0))

    def full_spec(shape):
        nd = len(shape)
        return pl.BlockSpec(shape, lambda h, _nd=nd: (0,) * _nd)

    in_specs = [
        head_spec(L, DH),
        head_spec(64, 32 * LORA), head_spec(64, 32 * LORA),
        head_spec(256, 8 * LORA), head_spec(256, 8 * LORA),
    ] + [full_spec(w.shape) for w in wts]
    return pl.pallas_call(
        _nsa_kernel,
        grid=(H,),
        in_specs=in_specs,
        out_specs=[head_spec(L, LORA), head_spec(L, LORA),
                   head_spec(1, 64), head_spec(1, 256)],
        out_shape=[jax.ShapeDtypeStruct((H, L, LORA), jnp.float32),
                   jax.ShapeDtypeStruct((H, L, LORA), jnp.float32),
                   jax.ShapeDtypeStruct((H, 1, 64), jnp.float32),
                   jax.ShapeDtypeStruct((H, 1, 256), jnp.float32)],
    )(q, kba, vba, kbb, vbb, *wts)


def _select(sca, scb, k2f, v2f):
    mesh = plsc.VectorSubcoreMesh(core_axis_name="c", subcore_axis_name="s")

    @functools.partial(
        pl.kernel, mesh=mesh,
        out_type=[
            jax.ShapeDtypeStruct((H, 16, 128), jnp.float32),
            jax.ShapeDtypeStruct((H, 16, 128), jnp.float32),
            jax.ShapeDtypeStruct((H, 16, 128), jnp.float32),
            jax.ShapeDtypeStruct((H, 16, 128), jnp.float32),
            jax.ShapeDtypeStruct((H, 16), jnp.int32),
            jax.ShapeDtypeStruct((H, 16), jnp.int32),
        ],
        scratch_types=[
            pltpu.VMEM((64,), jnp.float32),
            pltpu.VMEM((256,), jnp.float32),
            pltpu.VMEM((16,), jnp.int32),
            pltpu.VMEM((16, 128), jnp.float32),
            pltpu.VMEM((16, 128), jnp.float32),
            pltpu.VMEM((16,), jnp.int32),
            pltpu.SemaphoreType.DMA,
        ],
    )
    def sel_kernel(sa_hbm, sb_hbm, k2_hbm, v2_hbm,
                   ksa_hbm, vsa_hbm, ksb_hbm, vsb_hbm, ia_hbm, ib_hbm,
                   sa_v, sb_v, ridx_v, kr_v, vr_v, idx_v, sem):
        wid = lax.axis_index("s") * 2 + lax.axis_index("c")
        head = wid // 2
        scale = wid % 2
        iota = lax.iota(jnp.int32, 16)

        shuf_dn = lax.GatherDimensionNumbers(
            offset_dims=(), collapsed_slice_dims=(0,), start_index_map=(0,))

        def shuf(x, perm):
            return lax.gather(x, perm[:, None], shuf_dn, slice_sizes=(1,),
                              mode=lax.GatherScatterMode.PROMISE_IN_BOUNDS)

        def splat_max(x):
            for st in (1, 2, 4, 8):
                x = jnp.maximum(x, shuf(x, iota ^ st))
            return x

        def splat_min(x):
            for st in (1, 2, 4, 8):
                x = jnp.minimum(x, shuf(x, iota ^ st))
            return x

        def run(s_hbm, s_v, nblk, blk, topk, ks_hbm, vs_hbm, i_hbm):
            pltpu.sync_copy(s_hbm.at[head], s_v)
            nv = nblk // 16
            svs = [s_v[pl.ds(16 * j, 16)] for j in range(nv)]
            sel = jnp.zeros((16,), jnp.int32)
            inds = []
            for t in range(topk):
                vm = svs[0]
                for j in range(1, nv):
                    vm = jnp.maximum(vm, svs[j])
                mval = splat_max(vm)
                iv = jnp.full((16,), nblk, jnp.int32)
                for j in range(nv):
                    iv = jnp.minimum(
                        iv, jnp.where(svs[j] == mval, iota + 16 * j, nblk))
                ind = splat_min(iv)
                inds.append(ind)
                sel = jnp.where(iota == t, ind, sel)
                for j in range(nv):
                    svs[j] = jnp.where(iota + 16 * j == ind, NEG, svs[j])
            rpb = blk * LORA // 128
            slot = lax.shift_right_logical(iota, rpb.bit_length() - 1)
            rem = lax.bitwise_and(iota, rpb - 1)
            acc = jnp.zeros((16,), jnp.int32)
            for t in range(topk):
                acc = jnp.where(slot == t, inds[t] * rpb + rem, acc)
            ridx_v[...] = acc + head * (L * LORA // 128)
            idx_v[...] = sel
            pltpu.async_copy(k2_hbm.at[ridx_v], kr_v, sem).wait()
            pltpu.async_copy(v2_hbm.at[ridx_v], vr_v, sem).wait()
            pltpu.sync_copy(kr_v, ks_hbm.at[head])
            pltpu.sync_copy(vr_v, vs_hbm.at[head])
            pltpu.sync_copy(idx_v, i_hbm.at[head])

        @pl.when(scale == 0)
        def _():
            run(sa_hbm, sa_v, 64, 32, 2, ksa_hbm, vsa_hbm, ia_hbm)

        @pl.when(scale == 1)
        def _():
            run(sb_hbm, sb_v, 256, 8, 8, ksb_hbm, vsb_hbm, ib_hbm)

    return sel_kernel(sca, scb, k2f, v2f)


def _sel_scale(q2, ksel128, vsel128, idx_f, blk):
    kperm = jnp.concatenate(
        [ksel128[:, 32 * j:32 * (j + 1)] for j in range(4)], axis=0)
    vperm = jnp.concatenate(
        [vsel128[:, 32 * j:32 * (j + 1)] for j in range(4)], axis=0)
    rpb = blk * LORA // 128
    c = jax.lax.broadcasted_iota(jnp.int32, (64, 16), 0)
    slot = (c % 16) // rpb
    onehot = (slot == jax.lax.broadcasted_iota(jnp.int32, (64, 16), 1)
              ).astype(jnp.float32)
    bid = _dot_t(idx_f, onehot)
    qgrp = (jax.lax.broadcasted_iota(jnp.int32, (L, 1), 0) // blk
            ).astype(jnp.float32)
    selmask = jnp.where(bid <= qgrp, 0.0, NEG)
    logits = _dot_t(q2, kperm) / math.sqrt(LORA) + selmask
    w = _softmax(logits)
    return _dot_n(w, vperm)


QT = 1024


def _glob_kernel(q_ref, k_ref, v_ref, o_ref):
    i = pl.program_id(1)
    q = q_ref[0]
    row = jax.lax.broadcasted_iota(jnp.int32, (QT, QT), 0)
    col = jax.lax.broadcasted_iota(jnp.int32, (QT, QT), 1)

    def body(j, carry):
        m, l, acc = carry
        kt = k_ref[0, pl.ds(j * QT, QT), :]
        vt = v_ref[0, pl.ds(j * QT, QT), :]
        s = _dot_t(q, kt) / math.sqrt(DH)
        s = jnp.where(col + (j - i) * QT <= row, s, NEG)
        m2 = jnp.maximum(m, jnp.max(s, axis=-1, keepdims=True))
        p = jnp.exp(s - m2)
        sc = jnp.exp(m - m2)
        l2 = l * sc + jnp.sum(p, axis=-1, keepdims=True)
        acc2 = acc * sc + _dot_n(p, vt)
        return m2, l2, acc2

    m0 = jnp.full((QT, 1), NEG, jnp.float32)
    l0 = jnp.zeros((QT, 1), jnp.float32)
    a0 = jnp.zeros((QT, DH), jnp.float32)
    m, l, acc = jax.lax.fori_loop(0, i + 1, body, (m0, l0, a0))
    o_ref[0] = acc / l


def _glob(qkvh):
    return pl.pallas_call(
        _glob_kernel,
        grid=(H, L // QT),
        in_specs=[
            pl.BlockSpec((1, QT, DH), lambda h, i: (h, i, 0)),
            pl.BlockSpec((1, L, DH), lambda h, i: (H + h, 0, 0)),
            pl.BlockSpec((1, L, DH), lambda h, i: (2 * H + h, 0, 0)),
        ],
        out_specs=pl.BlockSpec((1, QT, DH), lambda h, i: (h, i, 0)),
        out_shape=jax.ShapeDtypeStruct((H, L, DH), jnp.float32),
    )(qkvh, qkvh, qkvh)


def _sel_att_kernel(q_ref, ksa_ref, vsa_ref, ksb_ref, vsb_ref,
                    ia_ref, ib_ref, sa_ref, sb_ref):
    q2 = q_ref[0][:, LORA:]
    sa_ref[0] = _sel_scale(q2, ksa_ref[0], vsa_ref[0],
                           ia_ref[0].astype(jnp.float32), 32)
    sb_ref[0] = _sel_scale(q2, ksb_ref[0], vsb_ref[0],
                           ib_ref[0].astype(jnp.float32), 8)


def _sel_att(q, ksa, vsa, ksb, vsb, ia, ib):
    def head_spec(n, d):
        return pl.BlockSpec((1, n, d), lambda h: (h, 0, 0))

    return pl.pallas_call(
        _sel_att_kernel,
        grid=(H,),
        in_specs=[head_spec(L, DH),
                  head_spec(16, 128), head_spec(16, 128),
                  head_spec(16, 128), head_spec(16, 128),
                  head_spec(1, 16), head_spec(1, 16)],
        out_specs=[head_spec(L, LORA), head_spec(L, LORA)],
        out_shape=[jax.ShapeDtypeStruct((H, L, LORA), jnp.float32),
                   jax.ShapeDtypeStruct((H, L, LORA), jnp.float32)],
    )(q, ksa, vsa, ksb, vsb, ia, ib)


def _merge_kernel(ca_ref, sa_ref, cb_ref, sb_ref, glb_ref, gate_ref,
                  g_ref, b_ref, wout_ref, o_ref):
    gt = gate_ref[...]
    pieces = []
    for h in range(H):
        g0 = gt[:, 3 * h:3 * h + 1]
        g1 = gt[:, 3 * h + 1:3 * h + 2]
        g2 = gt[:, 3 * h + 2:3 * h + 3]
        o1 = jnp.concatenate([ca_ref[h], sa_ref[h]], axis=1)
        o2 = jnp.concatenate([cb_ref[h], sb_ref[h]], axis=1)
        pieces.append(g0 * o1 + g1 * o2 + g2 * glb_ref[h])
    y = jnp.concatenate(pieces, axis=1)
    y = _layernorm(y, g_ref[...], b_ref[...])
    o_ref[...] = _dot_t(y, wout_ref[...])


def _merge(ca, sa, cb, sb, glb, gate, ln_g, ln_b, W_out):
    return pl.pallas_call(
        _merge_kernel,
        grid=(L // RT,),
        in_specs=[
            pl.BlockSpec((H, RT, LORA), lambda i: (0, i, 0)),
            pl.BlockSpec((H, RT, LORA), lambda i: (0, i, 0)),
            pl.BlockSpec((H, RT, LORA), lambda i: (0, i, 0)),
            pl.BlockSpec((H, RT, LORA), lambda i: (0, i, 0)),
            pl.BlockSpec((H, RT, DH), lambda i: (0, i, 0)),
            pl.BlockSpec((RT, 3 * H), lambda i: (i, 0)),
            pl.BlockSpec((1, E), lambda i: (0, 0)),
            pl.BlockSpec((1, E), lambda i: (0, 0)),
            pl.BlockSpec((E, E), lambda i: (0, 0)),
        ],
        out_specs=pl.BlockSpec((RT, E), lambda i: (i, 0)),
        out_shape=jax.ShapeDtypeStruct((L, E), jnp.float32),
    )(ca, sa, cb, sb, glb, gate, ln_g.reshape(1, E), ln_b.reshape(1, E),
      W_out)


def kernel(inp, ln_g, ln_b, W_qkv,
           a_kW1, a_kb1, a_kW2, a_kb2, a_vW1, a_vb1, a_vW2, a_vb2,
           b_kW1, b_kb1, b_kW2, b_kb2, b_vW1, b_vb1, b_vW2, b_vb2,
           W_gate, b_gate, W_out):
    qkv, gate = _pre(inp[0], ln_g, ln_b, W_qkv, W_gate, b_gate)
    qkvh = qkv.reshape(L, 3 * H, DH).transpose(1, 0, 2)
    k1 = qkvh[H:2 * H, :, :LORA]
    v1 = qkvh[2 * H:, :, :LORA]
    kba = k1.reshape(H, 64, 32 * LORA)
    vba = v1.reshape(H, 64, 32 * LORA)
    kbb = k1.reshape(H, 256, 8 * LORA)
    vbb = v1.reshape(H, 256, 8 * LORA)
    wts = (a_kW1, a_kb1.reshape(1, LORA), a_kW2, a_kb2.reshape(1, LORA),
           a_vW1, a_vb1.reshape(1, LORA), a_vW2, a_vb2.reshape(1, LORA),
           b_kW1, b_kb1.reshape(1, LORA), b_kW2, b_kb2.reshape(1, LORA),
           b_vW1, b_vb1.reshape(1, LORA), b_vW2, b_vb2.reshape(1, LORA))
    ca, cb, sca, scb = _nsa(qkvh, kba, vba, kbb, vbb, wts)
    k2f = qkvh[H:2 * H, :, LORA:].reshape(H * L * LORA // 128, 128)
    v2f = qkvh[2 * H:, :, LORA:].reshape(H * L * LORA // 128, 128)
    ksa, vsa, ksb, vsb, ia, ib = _select(
        sca.reshape(H, 64), scb.reshape(H, 256), k2f, v2f)
    glb = _glob(qkvh)
    sa, sb = _sel_att(qkvh, ksa, vsa, ksb, vsb,
                      ia.reshape(H, 1, 16), ib.reshape(H, 1, 16))
    out = _merge(ca, sa, cb, sb, glb, gate, ln_g, ln_b, W_out)
    return out.reshape(B, L, E)
